# Initial kernel scaffold; baseline (speedup 1.0000x reference)
import jax
import jax.numpy as jnp
from jax.experimental import pallas as pl


def kernel(z, lin_w, lin_b, bn1_g, bn1_b, ct1_w, ct1_b, bn2_g, bn2_b, ct2_w, ct2_b, bn3_g, bn3_b, cv_w, cv_b):
    raise NotImplementedError("write your pallas kernel here")


# speedup vs baseline: 1.8248x; 1.8248x over previous
import jax
import jax.numpy as jnp
from jax.experimental import pallas as pl
from jax.experimental.pallas import tpu as pltpu

EPS = 1e-5
NEG_SLOPE = 0.2
CH = 32
H0, W0 = 4, 19
LANES = 128
LIN_KSPLIT = 4


def _leaky(x):
    return jnp.where(x >= 0, x, NEG_SLOPE * x)


def _chan_stat(row):
    n = row.shape[-1]
    s = n // 2
    while s >= CH:
        row = row + pltpu.roll(row, s, axis=1)
        s //= 2
    return row * (CH / n)


def _lin_body(z_ref, w_ref, b_ref, o_ref):
    @pl.when(pl.program_id(0) == 0)
    def _():
        o_ref[...] = jnp.broadcast_to(b_ref[...], o_ref.shape)
    o_ref[...] += jnp.dot(z_ref[...], w_ref[...],
                          preferred_element_type=jnp.float32)


def _linear(z, w, b):
    B, K = z.shape
    N = w.shape[1]
    kb = K // LIN_KSPLIT
    return pl.pallas_call(
        _lin_body,
        out_shape=jax.ShapeDtypeStruct((B, N), jnp.float32),
        grid=(LIN_KSPLIT,),
        in_specs=[pl.BlockSpec((B, kb), lambda k: (0, k)),
                  pl.BlockSpec((kb, N), lambda k: (k, 0)),
                  pl.BlockSpec((1, N), lambda k: (0, 0))],
        out_specs=pl.BlockSpec((B, N), lambda k: (0, 0)),
        compiler_params=pltpu.CompilerParams(
            dimension_semantics=("arbitrary",)),
    )(z, w, b)


def _mid_body(x_ref, g1_ref, b1_ref, w1_ref, c1_ref, g2_ref, b2_ref,
              w2_ref, c2_ref, g3_ref, b3_ref, o_ref):
    x = x_ref[...]
    m1 = jnp.mean(x, axis=0, keepdims=True)
    d1 = x - m1
    v1 = jnp.mean(d1 * d1, axis=0, keepdims=True)
    xn = d1 * jax.lax.rsqrt(v1 + EPS) * g1_ref[...] + b1_ref[...]

    y1 = jnp.dot(xn, w1_ref[...], preferred_element_type=jnp.float32) \
        + c1_ref[...]

    m2 = _chan_stat(jnp.mean(y1, axis=0, keepdims=True))
    d2 = y1 - m2
    v2 = _chan_stat(jnp.mean(d2 * d2, axis=0, keepdims=True))
    y1a = _leaky(d2 * (jax.lax.rsqrt(v2 + EPS) * g2_ref[...]) + b2_ref[...])

    y2 = jnp.dot(y1a, w2_ref[...], preferred_element_type=jnp.float32) \
        + c2_ref[...]

    m3 = _chan_stat(jnp.mean(y2, axis=0, keepdims=True))
    d3 = y2 - m3
    v3 = _chan_stat(jnp.mean(d3 * d3, axis=0, keepdims=True))
    out = _leaky(d3 * (jax.lax.rsqrt(v3 + EPS) * g3_ref[...]) + b3_ref[...])
    o_ref[...] = out.astype(o_ref.dtype)


def _mid(x1, g1, b1, w1c, c1, g2, b2, w2d, c2, g3, b3):
    args = (x1, g1, b1, w1c, c1, g2, b2, w2d, c2, g3, b3)
    return pl.pallas_call(
        _mid_body,
        out_shape=jax.ShapeDtypeStruct((x1.shape[0], 16 * CH), jnp.bfloat16),
        in_specs=[pl.BlockSpec(memory_space=pltpu.MemorySpace.VMEM)] * len(args),
        out_specs=pl.BlockSpec(memory_space=pltpu.MemorySpace.VMEM),
        compiler_params=pltpu.CompilerParams(
            vmem_limit_bytes=56 * 1024 * 1024),
    )(*args)


WP = 80
NOUT = 256


def _conv_body(x_ref, w_ref, b_ref, o_ref):
    x = x_ref[...]
    p0 = jnp.dot(x, w_ref[0], preferred_element_type=jnp.float32)
    p1 = jnp.dot(x, w_ref[1], preferred_element_type=jnp.float32)
    p2 = jnp.dot(x, w_ref[2], preferred_element_type=jnp.float32)
    acc = pltpu.roll(p0, 1, axis=0) + p1 \
        + pltpu.roll(p2, x.shape[0] - 1, axis=0)
    o_ref[...] = jnp.tanh(acc + b_ref[...])


def _conv(x2, wt, cb):
    return pl.pallas_call(
        _conv_body,
        out_shape=jax.ShapeDtypeStruct((x2.shape[0], NOUT), jnp.float32),
        in_specs=[pl.BlockSpec(memory_space=pltpu.MemorySpace.VMEM)] * 3,
        out_specs=pl.BlockSpec(memory_space=pltpu.MemorySpace.VMEM),
        compiler_params=pltpu.CompilerParams(
            vmem_limit_bytes=56 * 1024 * 1024),
    )(x2, wt, cb)


def kernel(z, lin_w, lin_b, bn1_g, bn1_b, ct1_w, ct1_b, bn2_g, bn2_b,
           ct2_w, ct2_b, bn3_g, bn3_b, cv_w, cv_b):
    B = z.shape[0]

    w1c = jnp.transpose(ct1_w, (1, 0, 2)).reshape(CH, 4 * CH)
    c1 = jnp.tile(ct1_b, (1, 4))
    g2 = jnp.tile(bn2_g, (1, 4))
    b2 = jnp.tile(bn2_b, (1, 4))
    w2d = jnp.kron(jnp.eye(4, dtype=jnp.float32), ct2_w)
    c2 = jnp.tile(ct2_b, (1, 4))
    g3 = jnp.tile(bn3_g, (1, 4))
    b3 = jnp.tile(bn3_b, (1, 4))

    cw = cv_w[:, :, :3]
    wt = jnp.stack([
        sum(jnp.kron(jnp.eye(WP, k=1 - dx, dtype=jnp.float32),
                     cw[dy * 3 + dx]) for dx in range(3))
        for dy in range(3)])
    wt = jnp.pad(wt, ((0, 0), (0, 0), (0, NOUT - 3 * WP))).astype(jnp.bfloat16)
    cb = jnp.pad(jnp.tile(cv_b[:, :3], (1, WP)), ((0, 0), (0, NOUT - 3 * WP)))

    x = _linear(z, lin_w, lin_b)
    x1 = x.reshape(B * H0 * W0, CH)

    xa = _mid(x1, bn1_g, bn1_b, w1c, c1, g2, b2, w2d, c2, g3, b3)

    Hp = 4 * H0 + 2
    xr = xa.reshape(B, H0, W0, 2, 2, 2, 2, CH)
    xr = xr.transpose(0, 1, 3, 5, 2, 4, 6, 7).reshape(B, 4 * H0, 4 * W0 * CH)
    xp = jnp.pad(xr, ((0, 0), (1, 1), (CH, (WP - 1 - 4 * W0) * CH)))
    x2 = xp.reshape(B * Hp, WP * CH)

    y = _conv(x2, wt, cb)
    y = y[:, :3 * WP].reshape(B, Hp, WP, 3)[:, 1:Hp - 1, 1:1 + 4 * W0, :]
    return y.transpose(0, 3, 1, 2)


# speedup vs baseline: 3.7728x; 2.0675x over previous
import numpy as np

import jax
import jax.numpy as jnp
from jax.experimental import pallas as pl
from jax.experimental.pallas import tpu as pltpu

EPS = 1e-5
NEG_SLOPE = 0.2
CH = 32
H0, W0 = 4, 19
LANES = 128
LIN_KSPLIT = 8
WP = 80
NOUT = 256

_U = np.zeros((2560, CH), np.float32)
_U[np.arange(2560), np.arange(2560) % CH] = 1.0
_V = np.zeros((3, NOUT), np.float32)
_V[np.arange(3 * WP) % 3, np.arange(3 * WP)] = 1.0
_M = np.zeros((3, 2560, NOUT), np.float32)
for _dx in range(3):
    _M[_dx, :, :3 * WP] = np.kron(np.eye(WP, k=1 - _dx, dtype=np.float32),
                                  np.ones((CH, 3), np.float32))
_U = _U.astype(jnp.bfloat16)
_V = _V.astype(jnp.bfloat16)
_M = _M.astype(jnp.bfloat16)


def _leaky(x):
    return jnp.where(x >= 0, x, NEG_SLOPE * x)


def _chan_stat(row):
    n = row.shape[-1]
    s = n // 2
    while s >= CH:
        row = row + pltpu.roll(row, s, axis=1)
        s //= 2
    return row * (CH / n)


def _linmid_body(z_ref, w_ref, b_ref, g1_ref, b1_ref, w1_ref, c1_ref,
                 g2_ref, b2_ref, w2_ref, c2_ref, g3_ref, b3_ref,
                 o_ref, acc_ref):
    k = pl.program_id(0)

    @pl.when(k == 0)
    def _():
        acc_ref[...] = jnp.broadcast_to(jnp.transpose(b_ref[...]),
                                        acc_ref.shape)

    acc_ref[...] += jax.lax.dot_general(
        w_ref[...], z_ref[...], (((0,), (1,)), ((), ())),
        preferred_element_type=jnp.float32)

    @pl.when(k == LIN_KSPLIT - 1)
    def _():
        xt = acc_ref[...]
        nb = xt.shape[1]
        x = jnp.transpose(xt.reshape(H0 * W0, CH, nb), (0, 2, 1))
        x = x.reshape(H0 * W0 * nb, CH)

        m1 = jnp.mean(x, axis=0, keepdims=True)
        d1 = x - m1
        v1 = jnp.mean(d1 * d1, axis=0, keepdims=True)
        xn = d1 * jax.lax.rsqrt(v1 + EPS) * g1_ref[...] + b1_ref[...]

        w1c = jnp.concatenate([w1_ref[g] for g in range(4)], axis=1)
        y1 = jnp.dot(xn, w1c, preferred_element_type=jnp.float32) \
            + jnp.tile(c1_ref[...], (1, 4))

        m2 = _chan_stat(jnp.mean(y1, axis=0, keepdims=True))
        d2 = y1 - m2
        v2 = _chan_stat(jnp.mean(d2 * d2, axis=0, keepdims=True))
        s2 = jax.lax.rsqrt(v2 + EPS) * jnp.tile(g2_ref[...], (1, 4))
        y1a = _leaky(d2 * s2 + jnp.tile(b2_ref[...], (1, 4)))

        r = jax.lax.broadcasted_iota(jnp.int32, (4 * CH, 16 * CH), 0)
        c = jax.lax.broadcasted_iota(jnp.int32, (4 * CH, 16 * CH), 1)
        w2d = jnp.where(r // CH == c // (4 * CH),
                        jnp.tile(w2_ref[...], (4, 4)), 0.0)
        y2 = jnp.dot(y1a, w2d, preferred_element_type=jnp.float32) \
            + jnp.tile(c2_ref[...], (1, 4))

        m3 = _chan_stat(jnp.mean(y2, axis=0, keepdims=True))
        d3 = y2 - m3
        v3 = _chan_stat(jnp.mean(d3 * d3, axis=0, keepdims=True))
        s3 = jax.lax.rsqrt(v3 + EPS) * jnp.tile(g3_ref[...], (1, 4))
        out = _leaky(d3 * s3 + jnp.tile(b3_ref[...], (1, 4)))
        o_ref[...] = out.astype(o_ref.dtype)


def _linmid(z, lw, lb, g1, b1, w1, c1, g2, b2, w2, c2, g3, b3):
    B, K = z.shape
    N = lw.shape[1]
    kb = K // LIN_KSPLIT
    small = [g1, b1, w1, c1, g2, b2, w2, c2, g3, b3]
    return pl.pallas_call(
        _linmid_body,
        out_shape=jax.ShapeDtypeStruct((B * H0 * W0, 16 * CH), jnp.bfloat16),
        grid=(LIN_KSPLIT,),
        in_specs=[pl.BlockSpec((B, kb), lambda k: (0, k)),
                  pl.BlockSpec((kb, N), lambda k: (k, 0)),
                  pl.BlockSpec((1, N), lambda k: (0, 0))] +
                 [pl.BlockSpec(a.shape, lambda k, nd=a.ndim: (0,) * nd)
                  for a in small],
        out_specs=pl.BlockSpec((B * H0 * W0, 16 * CH), lambda k: (0, 0)),
        scratch_shapes=[pltpu.VMEM((N, B), jnp.float32)],
        compiler_params=pltpu.CompilerParams(
            dimension_semantics=("arbitrary",),
            vmem_limit_bytes=56 * 1024 * 1024),
    )(z, lw, lb, *small)


def _conv_body(x_ref, w_ref, b_ref, u_ref, v_ref, m_ref, o_ref):
    x = x_ref[...]
    acc = None
    for dy in range(3):
        wt = None
        for dx in range(3):
            cw = w_ref[dy * 3 + dx][:, :3].astype(jnp.bfloat16)
            t = jnp.dot(u_ref[...],
                        jnp.dot(cw, v_ref[...],
                                preferred_element_type=jnp.float32
                                ).astype(jnp.bfloat16),
                        preferred_element_type=jnp.float32).astype(jnp.bfloat16)
            piece = m_ref[dx] * t
            wt = piece if wt is None else wt + piece
        p = jnp.dot(x, wt, preferred_element_type=jnp.float32)
        if dy == 0:
            p = pltpu.roll(p, 1, axis=0)
        elif dy == 2:
            p = pltpu.roll(p, x.shape[0] - 1, axis=0)
        acc = p if acc is None else acc + p
    bias = jnp.tile(b_ref[:, :3], (1, WP))
    bias = jnp.pad(bias, ((0, 0), (0, NOUT - 3 * WP)))
    o_ref[...] = jnp.tanh(acc + bias)


def _conv(x2, cv_w, cv_b):
    args = (x2, cv_w, cv_b, _U, _V, _M)
    return pl.pallas_call(
        _conv_body,
        out_shape=jax.ShapeDtypeStruct((x2.shape[0], NOUT), jnp.float32),
        in_specs=[pl.BlockSpec(memory_space=pltpu.MemorySpace.VMEM)] * len(args),
        out_specs=pl.BlockSpec(memory_space=pltpu.MemorySpace.VMEM),
        compiler_params=pltpu.CompilerParams(
            vmem_limit_bytes=56 * 1024 * 1024),
    )(*args)


def kernel(z, lin_w, lin_b, bn1_g, bn1_b, ct1_w, ct1_b, bn2_g, bn2_b,
           ct2_w, ct2_b, bn3_g, bn3_b, cv_w, cv_b):
    B = z.shape[0]

    xa = _linmid(z, lin_w, lin_b, bn1_g, bn1_b, ct1_w, ct1_b,
                 bn2_g, bn2_b, ct2_w, ct2_b, bn3_g, bn3_b)

    Hp = 4 * H0 + 2
    xr = xa.reshape(H0, W0, B, 2, 2, 2, 2, CH)
    xr = xr.transpose(2, 0, 3, 5, 1, 4, 6, 7).reshape(B, 4 * H0, 4 * W0 * CH)
    xp = jnp.pad(xr, ((0, 0), (1, 1), (CH, (WP - 1 - 4 * W0) * CH)))
    x2 = xp.reshape(B * Hp, WP * CH)

    y = _conv(x2, cv_w, cv_b)
    y = y[:, :3 * WP].reshape(B, Hp, WP, 3)[:, 1:Hp - 1, 1:1 + 4 * W0, :]
    return y.transpose(0, 3, 1, 2)


# speedup vs baseline: 4.7592x; 1.2615x over previous
import numpy as np

import jax
import jax.numpy as jnp
from jax.experimental import pallas as pl
from jax.experimental.pallas import tpu as pltpu

EPS = 1e-5
NEG_SLOPE = 0.2
CH = 32
H0, W0 = 4, 19
LANES = 128
LIN_KSPLIT = 8
WP = 80
NOUT = 256

_U = np.zeros((2560, CH), np.float32)
_U[np.arange(2560), np.arange(2560) % CH] = 1.0
_V = np.zeros((3, NOUT), np.float32)
_V[np.arange(3 * WP) // WP, np.arange(3 * WP)] = 1.0
_M = np.zeros((3, 2560, NOUT), np.float32)
for _dx in range(3):
    _M[_dx, :, :3 * WP] = np.tile(
        np.kron(np.eye(WP, k=1 - _dx, dtype=np.float32),
                np.ones((CH, 1), np.float32)), (1, 3))
_U = _U.astype(jnp.bfloat16)
_V = _V.astype(jnp.bfloat16)
_M = _M.astype(jnp.bfloat16)


def _leaky(x):
    return jnp.where(x >= 0, x, NEG_SLOPE * x)


def _chan_stat(row):
    n = row.shape[-1]
    s = n // 2
    while s >= CH:
        row = row + pltpu.roll(row, s, axis=1)
        s //= 2
    return row * (CH / n)


def _linmid_body(z_ref, w_ref, b_ref, g1_ref, b1_ref, w1_ref, c1_ref,
                 g2_ref, b2_ref, w2_ref, c2_ref, g3_ref, b3_ref,
                 o_ref, acc_ref):
    k = pl.program_id(0)

    @pl.when(k == 0)
    def _():
        acc_ref[...] = jnp.broadcast_to(jnp.transpose(b_ref[...]),
                                        acc_ref.shape)

    acc_ref[...] += jax.lax.dot_general(
        w_ref[...], z_ref[...], (((0,), (1,)), ((), ())),
        preferred_element_type=jnp.float32)

    @pl.when(k == LIN_KSPLIT - 1)
    def _():
        xt = acc_ref[...]
        nb = xt.shape[1]
        x = jnp.transpose(xt.reshape(H0 * W0, CH, nb), (0, 2, 1))
        x = x.reshape(H0 * W0 * nb, CH)

        m1 = jnp.mean(x, axis=0, keepdims=True)
        d1 = x - m1
        v1 = jnp.mean(d1 * d1, axis=0, keepdims=True)
        xn = d1 * jax.lax.rsqrt(v1 + EPS) * g1_ref[...] + b1_ref[...]

        w1c = jnp.concatenate([w1_ref[g] for g in range(4)], axis=1)
        y1 = jnp.dot(xn, w1c, preferred_element_type=jnp.float32) \
            + jnp.tile(c1_ref[...], (1, 4))

        m2 = _chan_stat(jnp.mean(y1, axis=0, keepdims=True))
        d2 = y1 - m2
        v2 = _chan_stat(jnp.mean(d2 * d2, axis=0, keepdims=True))
        s2 = jax.lax.rsqrt(v2 + EPS) * jnp.tile(g2_ref[...], (1, 4))
        y1a = _leaky(d2 * s2 + jnp.tile(b2_ref[...], (1, 4)))

        r = jax.lax.broadcasted_iota(jnp.int32, (4 * CH, 16 * CH), 0)
        c = jax.lax.broadcasted_iota(jnp.int32, (4 * CH, 16 * CH), 1)
        w2d = jnp.where(r // CH == c // (4 * CH),
                        jnp.tile(w2_ref[...], (4, 4)), 0.0)
        y2 = jnp.dot(y1a, w2d, preferred_element_type=jnp.float32) \
            + jnp.tile(c2_ref[...], (1, 4))

        m3 = _chan_stat(jnp.mean(y2, axis=0, keepdims=True))
        d3 = y2 - m3
        v3 = _chan_stat(jnp.mean(d3 * d3, axis=0, keepdims=True))
        s3 = jax.lax.rsqrt(v3 + EPS) * jnp.tile(g3_ref[...], (1, 4))
        out = _leaky(d3 * s3 + jnp.tile(b3_ref[...], (1, 4)))
        o_ref[...] = out.astype(o_ref.dtype)


def _linmid(z, lw, lb, g1, b1, w1, c1, g2, b2, w2, c2, g3, b3):
    B, K = z.shape
    N = lw.shape[1]
    kb = K // LIN_KSPLIT
    small = [g1, b1, w1, c1, g2, b2, w2, c2, g3, b3]
    return pl.pallas_call(
        _linmid_body,
        out_shape=jax.ShapeDtypeStruct((B * H0 * W0, 16 * CH), jnp.bfloat16),
        grid=(LIN_KSPLIT,),
        in_specs=[pl.BlockSpec((B, kb), lambda k: (0, k)),
                  pl.BlockSpec((kb, N), lambda k: (k, 0)),
                  pl.BlockSpec((1, N), lambda k: (0, 0))] +
                 [pl.BlockSpec(a.shape, lambda k, nd=a.ndim: (0,) * nd)
                  for a in small],
        out_specs=pl.BlockSpec((B * H0 * W0, 16 * CH), lambda k: (0, 0)),
        scratch_shapes=[pltpu.VMEM((N, B), jnp.float32)],
        compiler_params=pltpu.CompilerParams(
            dimension_semantics=("arbitrary",),
            vmem_limit_bytes=56 * 1024 * 1024),
    )(z, lw, lb, *small)


def _conv_body(x_ref, w_ref, b_ref, u_ref, v_ref, m_ref, o_ref):
    x = x_ref[...]
    acc = None
    for dy in range(3):
        wt = None
        for dx in range(3):
            cw = w_ref[dy * 3 + dx][:, :3].astype(jnp.bfloat16)
            t = jnp.dot(u_ref[...],
                        jnp.dot(cw, v_ref[...],
                                preferred_element_type=jnp.float32
                                ).astype(jnp.bfloat16),
                        preferred_element_type=jnp.float32).astype(jnp.bfloat16)
            piece = m_ref[dx] * t
            wt = piece if wt is None else wt + piece
        p = jnp.dot(x, wt, preferred_element_type=jnp.float32)
        if dy == 0:
            p = pltpu.roll(p, 1, axis=0)
        elif dy == 2:
            p = pltpu.roll(p, x.shape[0] - 1, axis=0)
        acc = p if acc is None else acc + p
    bias = jnp.dot(b_ref[:, :3].astype(jnp.bfloat16), v_ref[...],
                   preferred_element_type=jnp.float32)
    s = jnp.tanh(acc + bias)
    nb = x.shape[0] // (4 * H0 + 2)
    for c in range(3):
        v = s[:, c * WP + 1:c * WP + 1 + 4 * W0]
        v = v.reshape(nb, 4 * H0 + 2, 4 * W0)[:, 1:4 * H0 + 1, :]
        o_ref[:, c, :, :] = v


def _conv(x2, cv_w, cv_b):
    nb = x2.shape[0] // (4 * H0 + 2)
    args = (x2, cv_w, cv_b, _U, _V, _M)
    return pl.pallas_call(
        _conv_body,
        out_shape=jax.ShapeDtypeStruct((nb, 3, 4 * H0, 4 * W0), jnp.float32),
        in_specs=[pl.BlockSpec(memory_space=pltpu.MemorySpace.VMEM)] * len(args),
        out_specs=pl.BlockSpec(memory_space=pltpu.MemorySpace.VMEM),
        compiler_params=pltpu.CompilerParams(
            vmem_limit_bytes=56 * 1024 * 1024),
    )(*args)


def kernel(z, lin_w, lin_b, bn1_g, bn1_b, ct1_w, ct1_b, bn2_g, bn2_b,
           ct2_w, ct2_b, bn3_g, bn3_b, cv_w, cv_b):
    B = z.shape[0]

    xa = _linmid(z, lin_w, lin_b, bn1_g, bn1_b, ct1_w, ct1_b,
                 bn2_g, bn2_b, ct2_w, ct2_b, bn3_g, bn3_b)

    Hp = 4 * H0 + 2
    xr = xa.reshape(H0, W0, B, 2, 2, 2, 2, CH)
    xr = xr.transpose(2, 0, 3, 5, 1, 4, 6, 7).reshape(B, 4 * H0, 4 * W0 * CH)
    xp = jnp.pad(xr, ((0, 0), (1, 1), (CH, (WP - 1 - 4 * W0) * CH)))
    x2 = xp.reshape(B * Hp, WP * CH)

    return _conv(x2, cv_w, cv_b)
